# baseline (speedup 1.0000x reference)
import functools

import jax
import jax.numpy as jnp
from jax.experimental import pallas as pl
from jax.experimental.pallas import tpu as pltpu

_EPS = 1e-6


def _stats_kernel(x_ref, o_ref, s_ref, q_ref, *, c):
    t_per_blk = x_ref.shape[0] // c
    n_chunks = c // 8

    for ts in range(t_per_blk):
        base = ts * c
        x0 = x_ref[pl.ds(base, 8), :]
        x1 = x_ref[pl.ds(base + 8, 8), :]
        s_ref[...] = x0 + x1
        q_ref[...] = x0 * x0 + x1 * x1

        def body(i, _, base=base):
            a = x_ref[pl.ds(base + i * 16, 8), :]
            b = x_ref[pl.ds(base + i * 16 + 8, 8), :]
            s_ref[...] += a + b
            q_ref[...] += a * a + b * b
            return 0

        jax.lax.fori_loop(1, n_chunks // 2, body, 0)
        o_ref[ts, 0] = jnp.sum(s_ref[...], axis=0, keepdims=True)[0]
        o_ref[ts, 1] = jnp.sum(q_ref[...], axis=0, keepdims=True)[0]


def _loss_kernel(st_ref, o_ref, *, n_oth_h, n_oth_w):
    t, _, hh, ww = st_ref.shape
    l = hh

    lin_h_rows, sq_h_rows, lin_w_rows, sq_w_rows = [], [], [], []
    for ti in range(t):
        s = st_ref[ti, 0]
        q = st_ref[ti, 1]
        lin_w_rows.append(jnp.sum(s, axis=0, keepdims=True))
        sq_w_rows.append(jnp.sum(q, axis=0, keepdims=True))
        lin_h_rows.append(jnp.sum(s, axis=1, keepdims=True).T)
        sq_h_rows.append(jnp.sum(q, axis=1, keepdims=True).T)
    lin_h = jnp.concatenate(lin_h_rows, axis=0)
    sq_h = jnp.concatenate(sq_h_rows, axis=0)
    lin_w = jnp.concatenate(lin_w_rows, axis=0)
    sq_w = jnp.concatenate(sq_w_rows, axis=0)

    r = jax.lax.broadcasted_iota(jnp.int32, (l, l), 0)
    cidx = jax.lax.broadcasted_iota(jnp.int32, (l, l), 1)
    m_suf = (r >= cidx).astype(jnp.float32)
    m_pre = (r <= cidx).astype(jnp.float32)
    idx = jax.lax.broadcasted_iota(jnp.int32, (1, l), 1).astype(jnp.float32)
    hi = jax.lax.Precision.HIGHEST

    def branch(sq, lin, n_oth):
        suf_sq = jnp.dot(sq, m_suf, precision=hi)
        suf_lin = jnp.dot(lin, m_suf, precision=hi)
        pre_sq = jnp.dot(sq, m_pre, precision=hi)
        pre_lin = jnp.dot(lin, m_pre, precision=hi)
        n_suf = (l - idx) * n_oth
        n_pre = (idx + 1.0) * n_oth
        ga_s = jnp.sqrt(suf_sq + (2.0 * _EPS) * suf_lin + (_EPS * _EPS) * n_suf)
        ga_p = jnp.sqrt(pre_sq + (2.0 * _EPS) * pre_lin + (_EPS * _EPS) * n_pre)
        return ga_s + ga_p

    g = branch(sq_h, lin_h, float(n_oth_h)) + branch(sq_w, lin_w, float(n_oth_w))
    per_i = jnp.dot(jnp.ones((1, t), jnp.float32), g, precision=hi)
    tot = jnp.sum(per_i, axis=1, keepdims=True)
    o_ref[...] = tot / (4.0 * t) + l * _EPS


def kernel(feat_map):
    t, c, h, w = feat_map.shape
    hw = h * w
    t_per_blk = 2 if t % 2 == 0 else 1
    xr = feat_map.reshape(t * c, hw)
    stats = pl.pallas_call(
        functools.partial(_stats_kernel, c=c),
        out_shape=jax.ShapeDtypeStruct((t, 2, hw), jnp.float32),
        grid=(t // t_per_blk,),
        in_specs=[pl.BlockSpec((t_per_blk * c, hw), lambda i: (i, 0))],
        out_specs=pl.BlockSpec((t_per_blk, 2, hw), lambda i: (i, 0, 0)),
        scratch_shapes=[
            pltpu.VMEM((8, hw), jnp.float32),
            pltpu.VMEM((8, hw), jnp.float32),
        ],
        compiler_params=pltpu.CompilerParams(
            dimension_semantics=("parallel",),
            vmem_limit_bytes=50 * 1024 * 1024,
        ),
        name="locality_stats",
    )(xr)
    out = pl.pallas_call(
        functools.partial(_loss_kernel, n_oth_h=c * w, n_oth_w=c * h),
        out_shape=jax.ShapeDtypeStruct((1, 1), jnp.float32),
        name="locality_loss_epilogue",
    )(stats.reshape(t, 2, h, w))
    return out[0, 0]


# speedup vs baseline: 8.6668x; 8.6668x over previous
import functools

import jax
import jax.numpy as jnp
from jax.experimental import pallas as pl
from jax.experimental.pallas import tpu as pltpu

_EPS = 1e-6
_FOLD = 128


def _fold_c(v, c):
    f = min(_FOLD, c)
    out = v[:, 0:f]
    for k in range(1, c // f):
        out = out + v[:, k * f:(k + 1) * f]
    return out


def _stats_kernel(x_ref, o_ref, aw_s_ref, aw_q_ref, bh_s_ref, bh_q_ref):
    _, h, w, c = x_ref.shape

    for hi in range(h):
        slab = x_ref[0, hi]
        sq = slab * slab
        fs = _fold_c(slab, c)
        fq = _fold_c(sq, c)
        if hi == 0:
            aw_s_ref[...] = fs
            aw_q_ref[...] = fq
        else:
            aw_s_ref[...] += fs
            aw_q_ref[...] += fq
        bh_s_ref[hi:hi + 1, :] = jnp.sum(fs, axis=0, keepdims=True)
        bh_q_ref[hi:hi + 1, :] = jnp.sum(fq, axis=0, keepdims=True)

    lin_h = jnp.sum(bh_s_ref[...], axis=1, keepdims=True).T
    sq_h = jnp.sum(bh_q_ref[...], axis=1, keepdims=True).T
    lin_w = jnp.sum(aw_s_ref[...], axis=1, keepdims=True).T
    sq_w = jnp.sum(aw_q_ref[...], axis=1, keepdims=True).T
    o_ref[0, 0] = lin_h[0]
    o_ref[0, 1] = sq_h[0]
    o_ref[0, 2] = lin_w[0]
    o_ref[0, 3] = sq_w[0]


def _loss_kernel(st_ref, o_ref, *, n_oth_h, n_oth_w):
    t, _, l = st_ref.shape
    lin_h = st_ref[:, 0, :]
    sq_h = st_ref[:, 1, :]
    lin_w = st_ref[:, 2, :]
    sq_w = st_ref[:, 3, :]

    r = jax.lax.broadcasted_iota(jnp.int32, (l, l), 0)
    cidx = jax.lax.broadcasted_iota(jnp.int32, (l, l), 1)
    m_suf = (r >= cidx).astype(jnp.float32)
    m_pre = (r <= cidx).astype(jnp.float32)
    idx = jax.lax.broadcasted_iota(jnp.int32, (1, l), 1).astype(jnp.float32)
    hi = jax.lax.Precision.HIGHEST

    def branch(sq, lin, n_oth):
        suf_sq = jnp.dot(sq, m_suf, precision=hi)
        suf_lin = jnp.dot(lin, m_suf, precision=hi)
        pre_sq = jnp.dot(sq, m_pre, precision=hi)
        pre_lin = jnp.dot(lin, m_pre, precision=hi)
        n_suf = (l - idx) * n_oth
        n_pre = (idx + 1.0) * n_oth
        ga_s = jnp.sqrt(suf_sq + (2.0 * _EPS) * suf_lin + (_EPS * _EPS) * n_suf)
        ga_p = jnp.sqrt(pre_sq + (2.0 * _EPS) * pre_lin + (_EPS * _EPS) * n_pre)
        return ga_s + ga_p

    g = branch(sq_h, lin_h, float(n_oth_h)) + branch(sq_w, lin_w, float(n_oth_w))
    per_i = jnp.dot(jnp.ones((1, t), jnp.float32), g, precision=hi)
    tot = jnp.sum(per_i, axis=1, keepdims=True)
    o_ref[...] = tot / (4.0 * t) + l * _EPS


def kernel(feat_map):
    t, c, h, w = feat_map.shape
    xt = feat_map.transpose(0, 2, 3, 1)
    stats = pl.pallas_call(
        _stats_kernel,
        out_shape=jax.ShapeDtypeStruct((t, 4, h), jnp.float32),
        grid=(t,),
        in_specs=[pl.BlockSpec((1, h, w, c), lambda i: (i, 0, 0, 0))],
        out_specs=pl.BlockSpec((1, 4, h), lambda i: (i, 0, 0)),
        scratch_shapes=[
            pltpu.VMEM((w, min(_FOLD, c)), jnp.float32),
            pltpu.VMEM((w, min(_FOLD, c)), jnp.float32),
            pltpu.VMEM((h, min(_FOLD, c)), jnp.float32),
            pltpu.VMEM((h, min(_FOLD, c)), jnp.float32),
        ],
        compiler_params=pltpu.CompilerParams(
            dimension_semantics=("parallel",),
            vmem_limit_bytes=50 * 1024 * 1024,
        ),
        name="locality_stats",
    )(xt)
    out = pl.pallas_call(
        functools.partial(_loss_kernel, n_oth_h=c * w, n_oth_w=c * h),
        out_shape=jax.ShapeDtypeStruct((1, 1), jnp.float32),
        name="locality_loss_epilogue",
    )(stats)
    return out[0, 0]


# speedup vs baseline: 9.4008x; 1.0847x over previous
import functools

import jax
import jax.numpy as jnp
from jax.experimental import pallas as pl
from jax.experimental.pallas import tpu as pltpu

_EPS = 1e-6
_FOLD = 128


def _fold_c(v, c):
    f = min(_FOLD, c)
    out = v[:, 0:f]
    for k in range(1, c // f):
        out = out + v[:, k * f:(k + 1) * f]
    return out


def _loss_from_stats(st, n_oth_h, n_oth_w):
    t, _, l = st.shape
    lin_h = st[:, 0, :]
    sq_h = st[:, 1, :]
    lin_w = st[:, 2, :]
    sq_w = st[:, 3, :]

    r = jax.lax.broadcasted_iota(jnp.int32, (l, l), 0)
    cidx = jax.lax.broadcasted_iota(jnp.int32, (l, l), 1)
    m_suf = (r >= cidx).astype(jnp.float32)
    m_pre = (r <= cidx).astype(jnp.float32)
    idx = jax.lax.broadcasted_iota(jnp.int32, (1, l), 1).astype(jnp.float32)
    hi = jax.lax.Precision.HIGHEST

    def branch(sq, lin, n_oth):
        suf_sq = jnp.dot(sq, m_suf, precision=hi)
        suf_lin = jnp.dot(lin, m_suf, precision=hi)
        pre_sq = jnp.dot(sq, m_pre, precision=hi)
        pre_lin = jnp.dot(lin, m_pre, precision=hi)
        n_suf = (l - idx) * n_oth
        n_pre = (idx + 1.0) * n_oth
        ga_s = jnp.sqrt(suf_sq + (2.0 * _EPS) * suf_lin + (_EPS * _EPS) * n_suf)
        ga_p = jnp.sqrt(pre_sq + (2.0 * _EPS) * pre_lin + (_EPS * _EPS) * n_pre)
        return ga_s + ga_p

    g = branch(sq_h, lin_h, n_oth_h) + branch(sq_w, lin_w, n_oth_w)
    per_i = jnp.dot(jnp.ones((1, t), jnp.float32), g, precision=hi)
    tot = jnp.sum(per_i, axis=1, keepdims=True)
    return tot / (4.0 * t) + l * _EPS


def _fused_kernel(x_ref, o_ref, aw_s_ref, aw_q_ref, bh_s_ref, bh_q_ref,
                  st_ref, *, n_grid, n_oth_h, n_oth_w):
    tpb, h, w, c = x_ref.shape
    i = pl.program_id(0)

    for ts in range(tpb):
        for hi in range(h):
            slab = x_ref[ts, hi]
            sq = slab * slab
            fs = _fold_c(slab, c)
            fq = _fold_c(sq, c)
            if hi == 0:
                aw_s_ref[...] = fs
                aw_q_ref[...] = fq
            else:
                aw_s_ref[...] += fs
                aw_q_ref[...] += fq
            bh_s_ref[hi:hi + 1, :] = jnp.sum(fs, axis=0, keepdims=True)
            bh_q_ref[hi:hi + 1, :] = jnp.sum(fq, axis=0, keepdims=True)

        lin_h = jnp.sum(bh_s_ref[...], axis=1, keepdims=True).T
        sq_h = jnp.sum(bh_q_ref[...], axis=1, keepdims=True).T
        lin_w = jnp.sum(aw_s_ref[...], axis=1, keepdims=True).T
        sq_w = jnp.sum(aw_q_ref[...], axis=1, keepdims=True).T
        st_ref[i * tpb + ts] = jnp.concatenate(
            [lin_h, sq_h, lin_w, sq_w], axis=0)

    @pl.when(i == n_grid - 1)
    def _():
        o_ref[...] = _loss_from_stats(
            st_ref[...], float(n_oth_h), float(n_oth_w))


def kernel(feat_map):
    t, c, h, w = feat_map.shape
    tpb = 2 if t % 2 == 0 else 1
    n_grid = t // tpb
    xt = feat_map.transpose(0, 2, 3, 1)
    fold = min(_FOLD, c)
    out = pl.pallas_call(
        functools.partial(_fused_kernel, n_grid=n_grid,
                          n_oth_h=c * w, n_oth_w=c * h),
        out_shape=jax.ShapeDtypeStruct((1, 1), jnp.float32),
        grid=(n_grid,),
        in_specs=[pl.BlockSpec((tpb, h, w, c), lambda i: (i, 0, 0, 0))],
        out_specs=pl.BlockSpec((1, 1), lambda i: (0, 0)),
        scratch_shapes=[
            pltpu.VMEM((w, fold), jnp.float32),
            pltpu.VMEM((w, fold), jnp.float32),
            pltpu.VMEM((h, fold), jnp.float32),
            pltpu.VMEM((h, fold), jnp.float32),
            pltpu.VMEM((t, 4, h), jnp.float32),
        ],
        compiler_params=pltpu.CompilerParams(
            dimension_semantics=("arbitrary",),
            vmem_limit_bytes=50 * 1024 * 1024,
        ),
        name="locality_loss_fused",
    )(xt)
    return out[0, 0]
